# baseline (speedup 1.0000x reference)
import functools

import jax
import jax.numpy as jnp
from jax import lax
from jax.experimental import pallas as pl
from jax.experimental.pallas import tpu as pltpu
from jax.experimental.pallas import tpu_sc as plsc

N = 10000
E = 320000
D = 128
H = D // 2
OUT = 128

NW = 32
EPT = E // NW
K = 80
NCH = EPT // K
OWN = 640
ZR = 80


def _splat(vec16, lane):
    idx = jnp.full((16, 1), lane, dtype=jnp.int32)
    return lax.gather(
        vec16, idx,
        lax.GatherDimensionNumbers(offset_dims=(), collapsed_slice_dims=(0,),
                                   start_index_map=(0,)),
        (1,), mode=lax.GatherScatterMode.PROMISE_IN_BOUNDS)


def _gate_body(x_ref, w_ref, o_ref):
    o_ref[...] = jnp.dot(x_ref[...], w_ref[...],
                         preferred_element_type=jnp.float32)


def _gate_proj(x2, gw2):
    blk = 2000
    return pl.pallas_call(
        _gate_body,
        grid=(N // blk,),
        in_specs=[
            pl.BlockSpec((blk, D), lambda i: (i, 0)),
            pl.BlockSpec((D, 2), lambda i: (0, 0)),
        ],
        out_specs=pl.BlockSpec((blk, 2), lambda i: (i, 0)),
        out_shape=jax.ShapeDtypeStruct((N, 2), jnp.float32),
    )(x2, gw2)


def _sc_body(xlo_hbm, xhi_hbm, src_hbm, dst_hbm, ew_hbm, gi_hbm, gj_hbm,
             gc_hbm, s_out, deg_out,
             src_v, dst_v, ew_v, coeff_v, gi_v, gj_v, gc_v,
             rows0_v, rows1_v, srcc_v, dstc_v, ones_v, zero_v, degz_v,
             S_sh, deg_sh, sem0, sem1, ssem0, ssem1):
    cid = lax.axis_index("c")
    sid = lax.axis_index("s")
    wid = sid * 2 + cid
    ebase = wid * EPT

    pltpu.async_copy(src_hbm.at[pl.ds(ebase, EPT)], src_v, sem0)
    pltpu.async_copy(dst_hbm.at[pl.ds(ebase, EPT)], dst_v, sem0)
    pltpu.async_copy(ew_hbm.at[pl.ds(ebase, EPT)], ew_v, sem0)
    pltpu.async_copy(gi_hbm, gi_v, sem0)
    pltpu.async_copy(gj_hbm, gj_v, sem0)
    pltpu.async_copy(gc_hbm, gc_v, sem0)

    z16 = jnp.zeros((16,), jnp.float32)
    p16 = jnp.where(lax.iota(jnp.int32, 16) == 0,
                    jnp.float32(1.0), jnp.float32(0.0))

    def _zv(i, c):
        zero_v[i // 4, pl.ds((i % 4) * 16, 16)] = z16
        return c
    lax.fori_loop(0, ZR * (H // 16), _zv, 0)

    def _dz(r, c):
        degz_v[r, pl.ds(0, 16)] = z16
        ones_v[r, pl.ds(0, 16)] = p16
        return c
    lax.fori_loop(0, ZR, _dz, 0)

    pltpu.make_async_copy(src_hbm.at[pl.ds(ebase, EPT)], src_v, sem0).wait()
    pltpu.make_async_copy(dst_hbm.at[pl.ds(ebase, EPT)], dst_v, sem0).wait()
    pltpu.make_async_copy(ew_hbm.at[pl.ds(ebase, EPT)], ew_v, sem0).wait()
    pltpu.make_async_copy(gi_hbm, gi_v, sem0).wait()
    pltpu.make_async_copy(gj_hbm, gj_v, sem0).wait()
    pltpu.make_async_copy(gc_hbm, gc_v, sem0).wait()

    gcv = gc_v[pl.ds(0, 16)]
    gwe = _splat(gcv, 0)
    gb = _splat(gcv, 1)

    def _coef(i, c):
        off = i * 16
        s16 = src_v[pl.ds(off, 16)]
        d16 = dst_v[pl.ds(off, 16)]
        gi = plsc.load_gather(gi_v, [d16])
        gj = plsc.load_gather(gj_v, [s16])
        w16 = ew_v[pl.ds(off, 16)]
        t = gi + gj + w16 * gwe + gb
        lamb = 1.0 / (1.0 + jnp.exp(-t))
        coeff_v[pl.ds(off, 16)] = w16 * lamb
        return c
    lax.fori_loop(0, EPT // 16, _coef, 0)

    row0 = sid * OWN
    nzc = jnp.where(sid == 15, (N - 15 * OWN) // ZR, OWN // ZR)

    for p in range(2):
        x_hbm = xlo_hbm if p == 0 else xhi_hbm

        def _zs(j, c):
            pltpu.sync_copy(zero_v, S_sh.at[pl.ds(row0 + j * ZR, ZR)])
            if p == 0:
                pltpu.sync_copy(degz_v, deg_sh.at[pl.ds(row0 + j * ZR, ZR)])
            return c
        lax.fori_loop(0, nzc, _zs, 0)

        plsc.subcore_barrier()

        def _fill(slot, ch):
            eoff = ch * K
            for g in range(K // 16):
                srcc_v[slot, pl.ds(g * 16, 16)] = (
                    src_v[pl.ds(eoff + g * 16, 16)])
                dstc_v[slot, pl.ds(g * 16, 16)] = (
                    dst_v[pl.ds(eoff + g * 16, 16)])

        def _scale(rows, ch):
            eoff = ch * K
            for g in range(K // 16):
                c16 = coeff_v[pl.ds(eoff + g * 16, 16)]
                for l in range(16):
                    e = g * 16 + l
                    sp = _splat(c16, l)
                    for col in range(H // 16):
                        cs = col * 16
                        rows[e, pl.ds(cs, 16)] = rows[e, pl.ds(cs, 16)] * sp

        def _sadd(rows, slot, ssem):
            pltpu.async_copy(rows, S_sh.at[dstc_v.at[slot]], ssem, add=True)
            if p == 0:
                pltpu.async_copy(ones_v, deg_sh.at[dstc_v.at[slot]], ssem,
                                 add=True)

        def _sdrain(rows, slot, ssem):
            pltpu.make_async_copy(rows, S_sh.at[dstc_v.at[slot]],
                                  ssem).wait()
            if p == 0:
                pltpu.make_async_copy(ones_v, deg_sh.at[dstc_v.at[slot]],
                                      ssem).wait()

        _fill(0, 0)
        pltpu.async_copy(x_hbm.at[srcc_v.at[0]], rows0_v, sem0)

        def _pair(i, c):
            c0 = 2 * i

            @pl.when(i > 0)
            def _():
                _sdrain(rows1_v, 1, ssem1)

            _fill(1, c0 + 1)
            pltpu.async_copy(x_hbm.at[srcc_v.at[1]], rows1_v, sem1)
            pltpu.make_async_copy(x_hbm.at[srcc_v.at[0]], rows0_v,
                                  sem0).wait()
            _scale(rows0_v, c0)
            _sadd(rows0_v, 0, ssem0)
            pltpu.make_async_copy(x_hbm.at[srcc_v.at[1]], rows1_v,
                                  sem1).wait()
            _scale(rows1_v, c0 + 1)
            _sadd(rows1_v, 1, ssem1)
            _sdrain(rows0_v, 0, ssem0)
            _fill(0, c0 + 2)
            pltpu.async_copy(x_hbm.at[srcc_v.at[0]], rows0_v, sem0)
            return c
        lax.fori_loop(0, (NCH - 1) // 2, _pair, 0)

        pltpu.make_async_copy(x_hbm.at[srcc_v.at[0]], rows0_v, sem0).wait()
        _scale(rows0_v, NCH - 1)
        _sadd(rows0_v, 0, ssem0)
        _sdrain(rows1_v, 1, ssem1)
        _sdrain(rows0_v, 0, ssem0)

        plsc.subcore_barrier()

        def _wo(j, c):
            r = row0 + j * ZR
            pltpu.sync_copy(S_sh.at[pl.ds(r, ZR)],
                            s_out.at[cid, p, pl.ds(r, ZR)])
            if p == 0:
                pltpu.sync_copy(deg_sh.at[pl.ds(r, ZR)],
                                deg_out.at[cid, pl.ds(r, ZR)])
            return c
        lax.fori_loop(0, nzc, _wo, 0)

        if p == 0:
            plsc.subcore_barrier()


def _sc_aggregate(xlo, xhi, src, dst, ew, gi, gj, gc):
    mesh = plsc.VectorSubcoreMesh(core_axis_name="c", subcore_axis_name="s")
    f = functools.partial(
        pl.kernel,
        mesh=mesh,
        compiler_params=pltpu.CompilerParams(needs_layout_passes=False,
                                             use_tc_tiling_on_sc=False),
        out_type=[
            jax.ShapeDtypeStruct((2, 2, N, H), jnp.float32),
            jax.ShapeDtypeStruct((2, N, 16), jnp.float32),
        ],
        scratch_types=[
            pltpu.VMEM((EPT,), jnp.int32),
            pltpu.VMEM((EPT,), jnp.int32),
            pltpu.VMEM((EPT,), jnp.float32),
            pltpu.VMEM((EPT,), jnp.float32),
            pltpu.VMEM((N,), jnp.float32),
            pltpu.VMEM((N,), jnp.float32),
            pltpu.VMEM((16,), jnp.float32),
            pltpu.VMEM((K, H), jnp.float32),
            pltpu.VMEM((K, H), jnp.float32),
            pltpu.VMEM((2, K), jnp.int32),
            pltpu.VMEM((2, K), jnp.int32),
            pltpu.VMEM((ZR, 16), jnp.float32),
            pltpu.VMEM((ZR, H), jnp.float32),
            pltpu.VMEM((ZR, 16), jnp.float32),
            pltpu.VMEM_SHARED((N, H), jnp.float32),
            pltpu.VMEM_SHARED((N, 16), jnp.float32),
            pltpu.SemaphoreType.DMA,
            pltpu.SemaphoreType.DMA,
            pltpu.SemaphoreType.DMA,
            pltpu.SemaphoreType.DMA,
        ],
    )(_sc_body)
    return f(xlo, xhi, src, dst, ew, gi, gj, gc)


def _final_body(x_ref, s_ref, d_ref, amp_ref, w_ref, b_ref, o_ref):
    s_lo = s_ref[0, 0] + s_ref[1, 0]
    s_hi = s_ref[0, 1] + s_ref[1, 1]
    s = jnp.concatenate([s_lo, s_hi], axis=-1)
    dg = jnp.maximum(d_ref[0, :, 0:1] + d_ref[1, :, 0:1], 1.0)
    aggr = s * amp_ref[...] / dg
    o_ref[...] = (jnp.dot(x_ref[...], w_ref[0:D, :],
                          preferred_element_type=jnp.float32)
                  + jnp.dot(aggr, w_ref[D:2 * D, :],
                            preferred_element_type=jnp.float32)
                  + b_ref[...])


def _final(x2, s_parts, deg_parts, amp_weight, sage_w, sage_b2):
    blk = 400
    return pl.pallas_call(
        _final_body,
        grid=(N // blk,),
        in_specs=[
            pl.BlockSpec((blk, D), lambda i: (i, 0)),
            pl.BlockSpec((2, 2, blk, H), lambda i: (0, 0, i, 0)),
            pl.BlockSpec((2, blk, 16), lambda i: (0, i, 0)),
            pl.BlockSpec((1, D), lambda i: (0, 0)),
            pl.BlockSpec((2 * D, OUT), lambda i: (0, 0)),
            pl.BlockSpec((1, OUT), lambda i: (0, 0)),
        ],
        out_specs=pl.BlockSpec((blk, OUT), lambda i: (i, 0)),
        out_shape=jax.ShapeDtypeStruct((N, OUT), jnp.float32),
    )(x2, s_parts, deg_parts, amp_weight, sage_w, sage_b2)


def kernel(X, edge_index, edge_weight, amp_weight, gate_w, gate_b, sage_w,
           sage_b):
    x2 = X[0]
    src = edge_index[0]
    dst = edge_index[1]
    gw2 = jnp.stack([gate_w[:D, 0], gate_w[D:2 * D, 0]], axis=1)
    gc = jnp.zeros((16,), jnp.float32)
    gc = gc.at[0].set(gate_w[2 * D, 0]).at[1].set(gate_b[0])

    g2 = _gate_proj(x2, gw2)
    gi = g2[:, 0]
    gj = g2[:, 1]

    xlo = x2[:, :H]
    xhi = x2[:, H:]
    s_parts, deg_parts = _sc_aggregate(xlo, xhi, src, dst, edge_weight,
                                       gi, gj, gc)

    out2 = _final(x2, s_parts, deg_parts, amp_weight, sage_w,
                  sage_b.reshape(1, OUT))
    return out2[None]


# speedup vs baseline: 1.0217x; 1.0217x over previous
import functools

import jax
import jax.numpy as jnp
from jax import lax
from jax.experimental import pallas as pl
from jax.experimental.pallas import tpu as pltpu
from jax.experimental.pallas import tpu_sc as plsc

N = 10000
E = 320000
D = 128
H = D // 2
OUT = 128

NW = 32
EPT = E // NW
K = 80
NCH = EPT // K
OWN = 640
ZR = 80


def _splat(vec16, lane):
    idx = jnp.full((16, 1), lane, dtype=jnp.int32)
    return lax.gather(
        vec16, idx,
        lax.GatherDimensionNumbers(offset_dims=(), collapsed_slice_dims=(0,),
                                   start_index_map=(0,)),
        (1,), mode=lax.GatherScatterMode.PROMISE_IN_BOUNDS)


def _gate_body(x_ref, w_ref, o_ref):
    o_ref[...] = jnp.dot(x_ref[...], w_ref[...],
                         preferred_element_type=jnp.float32)


def _gate_proj(x2, gw2):
    blk = 2000
    return pl.pallas_call(
        _gate_body,
        grid=(N // blk,),
        in_specs=[
            pl.BlockSpec((blk, D), lambda i: (i, 0)),
            pl.BlockSpec((D, 2), lambda i: (0, 0)),
        ],
        out_specs=pl.BlockSpec((blk, 2), lambda i: (i, 0)),
        out_shape=jax.ShapeDtypeStruct((N, 2), jnp.float32),
    )(x2, gw2)


def _sc_body(xlo_hbm, xhi_hbm, src_hbm, dst_hbm, ew_hbm, gi_hbm, gj_hbm,
             gc_hbm, zs_hbm, zd_hbm, on_hbm, s_out, deg_out,
             src_v, dst_v, ew_v, coeff_v, gi_v, gj_v, gc_v,
             rows0_v, rows1_v, rowsb0_v, rowsb1_v, srcc_v, dstc_v, ones_v,
             zero_v, degz_v, S_sh, deg_sh, sem0, sem1, ssem0, ssem1):
    cid = lax.axis_index("c")
    sid = lax.axis_index("s")
    wid = sid * 2 + cid
    ebase = wid * EPT

    pltpu.async_copy(src_hbm.at[pl.ds(ebase, EPT)], src_v, sem0)
    pltpu.async_copy(dst_hbm.at[pl.ds(ebase, EPT)], dst_v, sem0)
    pltpu.async_copy(ew_hbm.at[pl.ds(ebase, EPT)], ew_v, sem0)
    pltpu.async_copy(gi_hbm, gi_v, sem0)
    pltpu.async_copy(gj_hbm, gj_v, sem0)
    pltpu.async_copy(gc_hbm, gc_v, sem0)
    pltpu.async_copy(zs_hbm, zero_v, sem0)
    pltpu.async_copy(zd_hbm, degz_v, sem0)
    pltpu.async_copy(on_hbm, ones_v, sem0)

    pltpu.make_async_copy(src_hbm.at[pl.ds(ebase, EPT)], src_v, sem0).wait()
    pltpu.make_async_copy(dst_hbm.at[pl.ds(ebase, EPT)], dst_v, sem0).wait()
    pltpu.make_async_copy(ew_hbm.at[pl.ds(ebase, EPT)], ew_v, sem0).wait()
    pltpu.make_async_copy(gi_hbm, gi_v, sem0).wait()
    pltpu.make_async_copy(gj_hbm, gj_v, sem0).wait()
    pltpu.make_async_copy(gc_hbm, gc_v, sem0).wait()
    pltpu.make_async_copy(zs_hbm, zero_v, sem0).wait()
    pltpu.make_async_copy(zd_hbm, degz_v, sem0).wait()
    pltpu.make_async_copy(on_hbm, ones_v, sem0).wait()

    gcv = gc_v[pl.ds(0, 16)]
    gwe = _splat(gcv, 0)
    gb = _splat(gcv, 1)

    def _coef(i, c):
        off = i * 16
        s16 = src_v[pl.ds(off, 16)]
        d16 = dst_v[pl.ds(off, 16)]
        gi = plsc.load_gather(gi_v, [d16])
        gj = plsc.load_gather(gj_v, [s16])
        w16 = ew_v[pl.ds(off, 16)]
        t = gi + gj + w16 * gwe + gb
        lamb = 1.0 / (1.0 + jnp.exp(-t))
        coeff_v[pl.ds(off, 16)] = w16 * lamb
        return c
    lax.fori_loop(0, EPT // 16, _coef, 0)

    row0 = sid * OWN
    nzc = jnp.where(sid == 15, (N - 15 * OWN) // ZR, OWN // ZR)

    for p in range(2):
        x_hbm = xlo_hbm if p == 0 else xhi_hbm

        def _zs(j, c):
            pltpu.sync_copy(zero_v, S_sh.at[pl.ds(row0 + j * ZR, ZR)])
            if p == 0:
                pltpu.sync_copy(degz_v, deg_sh.at[pl.ds(row0 + j * ZR, ZR)])
            return c
        lax.fori_loop(0, nzc, _zs, 0)

        plsc.subcore_barrier()

        def _fill_src(slot, ch):
            eoff = ch * K
            for g in range(K // 16):
                srcc_v[slot, pl.ds(g * 16, 16)] = (
                    src_v[pl.ds(eoff + g * 16, 16)])

        def _fill_dst(slot, ch):
            eoff = ch * K
            for g in range(K // 16):
                dstc_v[slot, pl.ds(g * 16, 16)] = (
                    dst_v[pl.ds(eoff + g * 16, 16)])

        def _gather(slot, ch, rows_bf, gsem):
            _fill_src(slot, ch)
            pltpu.async_copy(x_hbm.at[srcc_v.at[slot]], rows_bf, gsem)

        def _scale(rows_bf, rows, ch):
            eoff = ch * K
            for g in range(K // 16):
                c16 = coeff_v[pl.ds(eoff + g * 16, 16)]
                for l in range(16):
                    e = g * 16 + l
                    sp = _splat(c16, l)
                    for c2 in range(H // 32):
                        b32 = rows_bf[e, pl.ds(c2 * 32, 32)]
                        v0, v1 = plsc.unpack(
                            b32, format=plsc.PackFormat.INTERLEAVED)
                        rows[e, pl.ds(c2 * 32, 16)] = v0 * sp
                        rows[e, pl.ds(c2 * 32 + 16, 16)] = v1 * sp

        def _sadd(rows, slot, ch, ssem):
            _fill_dst(slot, ch)
            pltpu.async_copy(rows, S_sh.at[dstc_v.at[slot]], ssem, add=True)
            if p == 0:
                pltpu.async_copy(ones_v, deg_sh.at[dstc_v.at[slot]], ssem,
                                 add=True)

        def _sdrain(rows, slot, ssem):
            pltpu.make_async_copy(rows, S_sh.at[dstc_v.at[slot]],
                                  ssem).wait()
            if p == 0:
                pltpu.make_async_copy(ones_v, deg_sh.at[dstc_v.at[slot]],
                                      ssem).wait()

        _gather(0, 0, rowsb0_v, sem0)

        def _pair(i, c):
            c0 = 2 * i

            @pl.when(i > 0)
            def _():
                _sdrain(rows1_v, 1, ssem1)

            _gather(1, c0 + 1, rowsb1_v, sem1)
            pltpu.make_async_copy(x_hbm.at[srcc_v.at[0]], rowsb0_v,
                                  sem0).wait()
            _scale(rowsb0_v, rows0_v, c0)
            _sadd(rows0_v, 0, c0, ssem0)
            _gather(0, c0 + 2, rowsb0_v, sem0)
            pltpu.make_async_copy(x_hbm.at[srcc_v.at[1]], rowsb1_v,
                                  sem1).wait()
            _scale(rowsb1_v, rows1_v, c0 + 1)
            _sadd(rows1_v, 1, c0 + 1, ssem1)
            _sdrain(rows0_v, 0, ssem0)
            return c
        lax.fori_loop(0, (NCH - 1) // 2, _pair, 0)

        pltpu.make_async_copy(x_hbm.at[srcc_v.at[0]], rowsb0_v, sem0).wait()
        _scale(rowsb0_v, rows0_v, NCH - 1)
        _sadd(rows0_v, 0, NCH - 1, ssem0)
        _sdrain(rows1_v, 1, ssem1)
        _sdrain(rows0_v, 0, ssem0)

        plsc.subcore_barrier()

        def _wo(j, c):
            r = row0 + j * ZR
            pltpu.sync_copy(S_sh.at[pl.ds(r, ZR)],
                            s_out.at[cid, p, pl.ds(r, ZR)])
            if p == 0:
                pltpu.sync_copy(deg_sh.at[pl.ds(r, ZR)],
                                deg_out.at[cid, pl.ds(r, ZR)])
            return c
        lax.fori_loop(0, nzc, _wo, 0)

        if p == 0:
            plsc.subcore_barrier()


def _sc_aggregate(xlo, xhi, src, dst, ew, gi, gj, gc, zs, zd, on):
    mesh = plsc.VectorSubcoreMesh(core_axis_name="c", subcore_axis_name="s")
    f = functools.partial(
        pl.kernel,
        mesh=mesh,
        compiler_params=pltpu.CompilerParams(needs_layout_passes=False,
                                             use_tc_tiling_on_sc=False),
        out_type=[
            jax.ShapeDtypeStruct((2, 2, N, H), jnp.float32),
            jax.ShapeDtypeStruct((2, N, 8), jnp.float32),
        ],
        scratch_types=[
            pltpu.VMEM((EPT,), jnp.int32),
            pltpu.VMEM((EPT,), jnp.int32),
            pltpu.VMEM((EPT,), jnp.float32),
            pltpu.VMEM((EPT,), jnp.float32),
            pltpu.VMEM((N,), jnp.float32),
            pltpu.VMEM((N,), jnp.float32),
            pltpu.VMEM((16,), jnp.float32),
            pltpu.VMEM((K, H), jnp.float32),
            pltpu.VMEM((K, H), jnp.float32),
            pltpu.VMEM((K, H), jnp.bfloat16),
            pltpu.VMEM((K, H), jnp.bfloat16),
            pltpu.VMEM((2, K), jnp.int32),
            pltpu.VMEM((2, K), jnp.int32),
            pltpu.VMEM((K, 8), jnp.float32),
            pltpu.VMEM((ZR, H), jnp.float32),
            pltpu.VMEM((ZR, 8), jnp.float32),
            pltpu.VMEM_SHARED((N, H), jnp.float32),
            pltpu.VMEM_SHARED((N, 8), jnp.float32),
            pltpu.SemaphoreType.DMA,
            pltpu.SemaphoreType.DMA,
            pltpu.SemaphoreType.DMA,
            pltpu.SemaphoreType.DMA,
        ],
    )(_sc_body)
    return f(xlo, xhi, src, dst, ew, gi, gj, gc, zs, zd, on)


def _final_body(x_ref, s_ref, d_ref, amp_ref, w_ref, b_ref, o_ref):
    s_lo = s_ref[0, 0] + s_ref[1, 0]
    s_hi = s_ref[0, 1] + s_ref[1, 1]
    s = jnp.concatenate([s_lo, s_hi], axis=-1)
    dg = jnp.maximum(d_ref[0, :, 0:1] + d_ref[1, :, 0:1], 1.0)
    aggr = s * amp_ref[...] / dg
    o_ref[...] = (jnp.dot(x_ref[...], w_ref[0:D, :],
                          preferred_element_type=jnp.float32)
                  + jnp.dot(aggr, w_ref[D:2 * D, :],
                            preferred_element_type=jnp.float32)
                  + b_ref[...])


def _final(x2, s_parts, deg_parts, amp_weight, sage_w, sage_b2):
    blk = 400
    return pl.pallas_call(
        _final_body,
        grid=(N // blk,),
        in_specs=[
            pl.BlockSpec((blk, D), lambda i: (i, 0)),
            pl.BlockSpec((2, 2, blk, H), lambda i: (0, 0, i, 0)),
            pl.BlockSpec((2, blk, 8), lambda i: (0, i, 0)),
            pl.BlockSpec((1, D), lambda i: (0, 0)),
            pl.BlockSpec((2 * D, OUT), lambda i: (0, 0)),
            pl.BlockSpec((1, OUT), lambda i: (0, 0)),
        ],
        out_specs=pl.BlockSpec((blk, OUT), lambda i: (i, 0)),
        out_shape=jax.ShapeDtypeStruct((N, OUT), jnp.float32),
    )(x2, s_parts, deg_parts, amp_weight, sage_w, sage_b2)


def kernel(X, edge_index, edge_weight, amp_weight, gate_w, gate_b, sage_w,
           sage_b):
    x2 = X[0]
    src = edge_index[0]
    dst = edge_index[1]
    gw2 = jnp.stack([gate_w[:D, 0], gate_w[D:2 * D, 0]], axis=1)
    gc = jnp.zeros((16,), jnp.float32)
    gc = gc.at[0].set(gate_w[2 * D, 0]).at[1].set(gate_b[0])

    g2 = _gate_proj(x2, gw2)
    gi = g2[:, 0]
    gj = g2[:, 1]

    def _prep(xh):
        a = xh.reshape(N, H // 32, 2, 16)
        return a.transpose(0, 1, 3, 2).reshape(N, H).astype(jnp.bfloat16)

    xlo = _prep(x2[:, :H])
    xhi = _prep(x2[:, H:])
    zs = jnp.zeros((ZR, H), jnp.float32)
    zd = jnp.zeros((ZR, 8), jnp.float32)
    on = jnp.zeros((K, 8), jnp.float32).at[:, 0].set(1.0)
    s_parts, deg_parts = _sc_aggregate(xlo, xhi, src, dst, edge_weight,
                                       gi, gj, gc, zs, zd, on)

    out2 = _final(x2, s_parts, deg_parts, amp_weight, sage_w,
                  sage_b.reshape(1, OUT))
    return out2[None]


# speedup vs baseline: 1.0221x; 1.0004x over previous
import functools

import jax
import jax.numpy as jnp
from jax import lax
from jax.experimental import pallas as pl
from jax.experimental.pallas import tpu as pltpu
from jax.experimental.pallas import tpu_sc as plsc

N = 10000
E = 320000
D = 128
H = D // 2
OUT = 128

NW = 32
EPT = E // NW
K = 80
NCH = EPT // K
OWN = 640
ZR = 80


def _splat(vec16, lane):
    idx = jnp.full((16, 1), lane, dtype=jnp.int32)
    return lax.gather(
        vec16, idx,
        lax.GatherDimensionNumbers(offset_dims=(), collapsed_slice_dims=(0,),
                                   start_index_map=(0,)),
        (1,), mode=lax.GatherScatterMode.PROMISE_IN_BOUNDS)


def _gate_body(x_ref, w_ref, o_ref):
    o_ref[...] = jnp.dot(x_ref[...], w_ref[...],
                         preferred_element_type=jnp.float32)


def _gate_proj(x2, gw2):
    blk = 2000
    return pl.pallas_call(
        _gate_body,
        grid=(N // blk,),
        in_specs=[
            pl.BlockSpec((blk, D), lambda i: (i, 0)),
            pl.BlockSpec((D, 2), lambda i: (0, 0)),
        ],
        out_specs=pl.BlockSpec((blk, 2), lambda i: (i, 0)),
        out_shape=jax.ShapeDtypeStruct((N, 2), jnp.float32),
    )(x2, gw2)


def _sc_body(xlo_hbm, xhi_hbm, src_hbm, dst_hbm, ew_hbm, gi_hbm, gj_hbm,
             gc_hbm, zs_hbm, zd_hbm, on_hbm, s_out, deg_out,
             src_v, dst_v, ew_v, coeff_v, gi_v, gj_v, gc_v,
             rows0_v, rows1_v, rowsb0_v, rowsb1_v, srcc_v, dstc_v, ones_v,
             zero_v, degz_v, S_sh, deg_sh, sem0, sem1, ssem0, ssem1):
    cid = lax.axis_index("c")
    sid = lax.axis_index("s")
    wid = sid * 2 + cid
    ebase = wid * EPT

    pltpu.async_copy(src_hbm.at[pl.ds(ebase, EPT)], src_v, sem0)
    pltpu.async_copy(dst_hbm.at[pl.ds(ebase, EPT)], dst_v, sem0)
    pltpu.async_copy(ew_hbm.at[pl.ds(ebase, EPT)], ew_v, sem0)
    pltpu.async_copy(gi_hbm, gi_v, sem0)
    pltpu.async_copy(gj_hbm, gj_v, sem0)
    pltpu.async_copy(gc_hbm, gc_v, sem0)
    pltpu.async_copy(zs_hbm, zero_v, sem0)
    pltpu.async_copy(zd_hbm, degz_v, sem0)
    pltpu.async_copy(on_hbm, ones_v, sem0)

    pltpu.make_async_copy(src_hbm.at[pl.ds(ebase, EPT)], src_v, sem0).wait()
    pltpu.make_async_copy(dst_hbm.at[pl.ds(ebase, EPT)], dst_v, sem0).wait()
    pltpu.make_async_copy(ew_hbm.at[pl.ds(ebase, EPT)], ew_v, sem0).wait()
    pltpu.make_async_copy(gi_hbm, gi_v, sem0).wait()
    pltpu.make_async_copy(gj_hbm, gj_v, sem0).wait()
    pltpu.make_async_copy(gc_hbm, gc_v, sem0).wait()
    pltpu.make_async_copy(zs_hbm, zero_v, sem0).wait()
    pltpu.make_async_copy(zd_hbm, degz_v, sem0).wait()
    pltpu.make_async_copy(on_hbm, ones_v, sem0).wait()

    with jax.named_scope("coef"):
        gcv = gc_v[pl.ds(0, 16)]
        gwe = _splat(gcv, 0)
        gb = _splat(gcv, 1)

        def _coef(i, c):
            off = i * 16
            s16 = src_v[pl.ds(off, 16)]
            d16 = dst_v[pl.ds(off, 16)]
            gi = plsc.load_gather(gi_v, [d16])
            gj = plsc.load_gather(gj_v, [s16])
            w16 = ew_v[pl.ds(off, 16)]
            t = gi + gj + w16 * gwe + gb
            lamb = 1.0 / (1.0 + jnp.exp(-t))
            coeff_v[pl.ds(off, 16)] = w16 * lamb
            return c
        lax.fori_loop(0, EPT // 16, _coef, 0)

    row0 = sid * OWN
    nzc = jnp.where(sid == 15, (N - 15 * OWN) // ZR, OWN // ZR)

    for p in range(2):
        x_hbm = xlo_hbm if p == 0 else xhi_hbm

        with jax.named_scope(f"zero{p}"):
            def _zs(j, c):
                pltpu.sync_copy(zero_v, S_sh.at[pl.ds(row0 + j * ZR, ZR)])
                if p == 0:
                    pltpu.sync_copy(degz_v,
                                    deg_sh.at[pl.ds(row0 + j * ZR, ZR)])
                return c
            lax.fori_loop(0, nzc, _zs, 0)

            plsc.subcore_barrier()

        def _fill_src(slot, ch):
            eoff = ch * K
            for g in range(K // 16):
                srcc_v[slot, pl.ds(g * 16, 16)] = (
                    src_v[pl.ds(eoff + g * 16, 16)])

        def _fill_dst(slot, ch):
            eoff = ch * K
            for g in range(K // 16):
                dstc_v[slot, pl.ds(g * 16, 16)] = (
                    dst_v[pl.ds(eoff + g * 16, 16)])

        def _gather(slot, ch, rows_bf, gsem):
            _fill_src(slot, ch)
            pltpu.async_copy(x_hbm.at[srcc_v.at[slot]], rows_bf, gsem)

        def _scale(rows_bf, rows, ch):
            eoff = ch * K
            for g in range(K // 16):
                c16 = coeff_v[pl.ds(eoff + g * 16, 16)]
                for l in range(16):
                    e = g * 16 + l
                    sp = _splat(c16, l)
                    for c2 in range(H // 32):
                        b32 = rows_bf[e, pl.ds(c2 * 32, 32)]
                        v0, v1 = plsc.unpack(
                            b32, format=plsc.PackFormat.INTERLEAVED)
                        rows[e, pl.ds(c2 * 32, 16)] = v0 * sp
                        rows[e, pl.ds(c2 * 32 + 16, 16)] = v1 * sp

        def _sadd(rows, slot, ch, ssem):
            _fill_dst(slot, ch)
            pltpu.async_copy(rows, S_sh.at[dstc_v.at[slot]], ssem, add=True)
            if p == 0:
                pltpu.async_copy(ones_v, deg_sh.at[dstc_v.at[slot]], ssem,
                                 add=True)

        def _sdrain(rows, slot, ssem):
            pltpu.make_async_copy(rows, S_sh.at[dstc_v.at[slot]],
                                  ssem).wait()
            if p == 0:
                pltpu.make_async_copy(ones_v, deg_sh.at[dstc_v.at[slot]],
                                      ssem).wait()

        sc_main = jax.named_scope(f"main{p}")
        sc_main.__enter__()
        _gather(0, 0, rowsb0_v, sem0)

        def _pair(i, c):
            c0 = 2 * i

            @pl.when(i > 0)
            def _():
                _sdrain(rows1_v, 1, ssem1)

            _gather(1, c0 + 1, rowsb1_v, sem1)
            pltpu.make_async_copy(x_hbm.at[srcc_v.at[0]], rowsb0_v,
                                  sem0).wait()
            _scale(rowsb0_v, rows0_v, c0)
            _sadd(rows0_v, 0, c0, ssem0)
            _gather(0, c0 + 2, rowsb0_v, sem0)
            pltpu.make_async_copy(x_hbm.at[srcc_v.at[1]], rowsb1_v,
                                  sem1).wait()
            _scale(rowsb1_v, rows1_v, c0 + 1)
            _sadd(rows1_v, 1, c0 + 1, ssem1)
            _sdrain(rows0_v, 0, ssem0)
            return c
        lax.fori_loop(0, (NCH - 1) // 2, _pair, 0)

        pltpu.make_async_copy(x_hbm.at[srcc_v.at[0]], rowsb0_v, sem0).wait()
        _scale(rowsb0_v, rows0_v, NCH - 1)
        _sadd(rows0_v, 0, NCH - 1, ssem0)
        _sdrain(rows1_v, 1, ssem1)
        _sdrain(rows0_v, 0, ssem0)
        sc_main.__exit__(None, None, None)

        with jax.named_scope(f"wout{p}"):
            plsc.subcore_barrier()

            def _wo(j, c):
                r = row0 + j * ZR
                pltpu.sync_copy(S_sh.at[pl.ds(r, ZR)],
                                s_out.at[cid, p, pl.ds(r, ZR)])
                if p == 0:
                    pltpu.sync_copy(deg_sh.at[pl.ds(r, ZR)],
                                    deg_out.at[cid, pl.ds(r, ZR)])
                return c
            lax.fori_loop(0, nzc, _wo, 0)

            if p == 0:
                plsc.subcore_barrier()


def _sc_aggregate(xlo, xhi, src, dst, ew, gi, gj, gc, zs, zd, on):
    mesh = plsc.VectorSubcoreMesh(core_axis_name="c", subcore_axis_name="s")
    f = functools.partial(
        pl.kernel,
        mesh=mesh,
        compiler_params=pltpu.CompilerParams(needs_layout_passes=False,
                                             use_tc_tiling_on_sc=False),
        out_type=[
            jax.ShapeDtypeStruct((2, 2, N, H), jnp.float32),
            jax.ShapeDtypeStruct((2, N, 8), jnp.float32),
        ],
        scratch_types=[
            pltpu.VMEM((EPT,), jnp.int32),
            pltpu.VMEM((EPT,), jnp.int32),
            pltpu.VMEM((EPT,), jnp.float32),
            pltpu.VMEM((EPT,), jnp.float32),
            pltpu.VMEM((N,), jnp.float32),
            pltpu.VMEM((N,), jnp.float32),
            pltpu.VMEM((16,), jnp.float32),
            pltpu.VMEM((K, H), jnp.float32),
            pltpu.VMEM((K, H), jnp.float32),
            pltpu.VMEM((K, H), jnp.bfloat16),
            pltpu.VMEM((K, H), jnp.bfloat16),
            pltpu.VMEM((2, K), jnp.int32),
            pltpu.VMEM((2, K), jnp.int32),
            pltpu.VMEM((K, 8), jnp.float32),
            pltpu.VMEM((ZR, H), jnp.float32),
            pltpu.VMEM((ZR, 8), jnp.float32),
            pltpu.VMEM_SHARED((N, H), jnp.float32),
            pltpu.VMEM_SHARED((N, 8), jnp.float32),
            pltpu.SemaphoreType.DMA,
            pltpu.SemaphoreType.DMA,
            pltpu.SemaphoreType.DMA,
            pltpu.SemaphoreType.DMA,
        ],
    )(_sc_body)
    return f(xlo, xhi, src, dst, ew, gi, gj, gc, zs, zd, on)


def _final_body(x_ref, s_ref, d_ref, amp_ref, w_ref, b_ref, o_ref):
    s_lo = s_ref[0, 0] + s_ref[1, 0]
    s_hi = s_ref[0, 1] + s_ref[1, 1]
    s = jnp.concatenate([s_lo, s_hi], axis=-1)
    dg = jnp.maximum(d_ref[0, :, 0:1] + d_ref[1, :, 0:1], 1.0)
    aggr = s * amp_ref[...] / dg
    o_ref[...] = (jnp.dot(x_ref[...], w_ref[0:D, :],
                          preferred_element_type=jnp.float32)
                  + jnp.dot(aggr, w_ref[D:2 * D, :],
                            preferred_element_type=jnp.float32)
                  + b_ref[...])


def _final(x2, s_parts, deg_parts, amp_weight, sage_w, sage_b2):
    blk = 400
    return pl.pallas_call(
        _final_body,
        grid=(N // blk,),
        in_specs=[
            pl.BlockSpec((blk, D), lambda i: (i, 0)),
            pl.BlockSpec((2, 2, blk, H), lambda i: (0, 0, i, 0)),
            pl.BlockSpec((2, blk, 8), lambda i: (0, i, 0)),
            pl.BlockSpec((1, D), lambda i: (0, 0)),
            pl.BlockSpec((2 * D, OUT), lambda i: (0, 0)),
            pl.BlockSpec((1, OUT), lambda i: (0, 0)),
        ],
        out_specs=pl.BlockSpec((blk, OUT), lambda i: (i, 0)),
        out_shape=jax.ShapeDtypeStruct((N, OUT), jnp.float32),
    )(x2, s_parts, deg_parts, amp_weight, sage_w, sage_b2)


def kernel(X, edge_index, edge_weight, amp_weight, gate_w, gate_b, sage_w,
           sage_b):
    x2 = X[0]
    src = edge_index[0]
    dst = edge_index[1]
    gw2 = jnp.stack([gate_w[:D, 0], gate_w[D:2 * D, 0]], axis=1)
    gc = jnp.zeros((16,), jnp.float32)
    gc = gc.at[0].set(gate_w[2 * D, 0]).at[1].set(gate_b[0])

    g2 = _gate_proj(x2, gw2)
    gi = g2[:, 0]
    gj = g2[:, 1]

    def _prep(xh):
        a = xh.reshape(N, H // 32, 2, 16)
        return a.transpose(0, 1, 3, 2).reshape(N, H).astype(jnp.bfloat16)

    xlo = _prep(x2[:, :H])
    xhi = _prep(x2[:, H:])
    zs = jnp.zeros((ZR, H), jnp.float32)
    zd = jnp.zeros((ZR, 8), jnp.float32)
    on = jnp.zeros((K, 8), jnp.float32).at[:, 0].set(1.0)
    s_parts, deg_parts = _sc_aggregate(xlo, xhi, src, dst, edge_weight,
                                       gi, gj, gc, zs, zd, on)

    out2 = _final(x2, s_parts, deg_parts, amp_weight, sage_w,
                  sage_b.reshape(1, OUT))
    return out2[None]


# speedup vs baseline: 1.1249x; 1.1006x over previous
import functools

import jax
import jax.numpy as jnp
from jax import lax
from jax.experimental import pallas as pl
from jax.experimental.pallas import tpu as pltpu
from jax.experimental.pallas import tpu_sc as plsc

N = 10000
E = 320000
D = 128
H = D // 2
OUT = 128

NW = 32
EPT = E // NW
K = 80
NCH = EPT // K
OWN = 640
ZR = 80


def _splat(vec16, lane):
    idx = jnp.full((16, 1), lane, dtype=jnp.int32)
    return lax.gather(
        vec16, idx,
        lax.GatherDimensionNumbers(offset_dims=(), collapsed_slice_dims=(0,),
                                   start_index_map=(0,)),
        (1,), mode=lax.GatherScatterMode.PROMISE_IN_BOUNDS)


def _gate_body(x_ref, w_ref, o_ref):
    o_ref[...] = jnp.dot(x_ref[...], w_ref[...],
                         preferred_element_type=jnp.float32)


def _gate_proj(x2, gw2):
    blk = 2000
    return pl.pallas_call(
        _gate_body,
        grid=(N // blk,),
        in_specs=[
            pl.BlockSpec((blk, D), lambda i: (i, 0)),
            pl.BlockSpec((D, 2), lambda i: (0, 0)),
        ],
        out_specs=pl.BlockSpec((blk, 2), lambda i: (i, 0)),
        out_shape=jax.ShapeDtypeStruct((N, 2), jnp.float32),
    )(x2, gw2)


def _sc_body(xlo_hbm, xhi_hbm, src_hbm, dst_hbm, ew_hbm, gi_hbm, gj_hbm,
             gc_hbm, zs_hbm, zd_hbm, on_hbm, s_out, deg_out,
             src_v, dst_v, ew_v, coeff_v, gi_v, gj_v, gc_v,
             rw0, rw1, rb0, rb1, srcc_v, dstc_v, ones_v,
             zero_v, degz_v, S_sh, deg_sh,
             gs0, gs1, ss0, ss1):
    gsems = [gs0, gs1]
    ssems = [ss0, ss1]
    rows_l = [rw0, rw1]
    rowsb_l = [rb0, rb1]
    cid = lax.axis_index("c")
    sid = lax.axis_index("s")
    wid = sid * 2 + cid
    ebase = wid * EPT

    sem0 = gsems[0]
    pltpu.async_copy(src_hbm.at[pl.ds(ebase, EPT)], src_v, sem0)
    pltpu.async_copy(dst_hbm.at[pl.ds(ebase, EPT)], dst_v, sem0)
    pltpu.async_copy(ew_hbm.at[pl.ds(ebase, EPT)], ew_v, sem0)
    pltpu.async_copy(gi_hbm, gi_v, sem0)
    pltpu.async_copy(gj_hbm, gj_v, sem0)
    pltpu.async_copy(gc_hbm, gc_v, sem0)
    pltpu.async_copy(zs_hbm, zero_v, sem0)
    pltpu.async_copy(zd_hbm, degz_v, sem0)
    pltpu.async_copy(on_hbm, ones_v, sem0)

    pltpu.make_async_copy(src_hbm.at[pl.ds(ebase, EPT)], src_v,
                          sem0).wait()
    pltpu.make_async_copy(dst_hbm.at[pl.ds(ebase, EPT)], dst_v,
                          sem0).wait()
    pltpu.make_async_copy(ew_hbm.at[pl.ds(ebase, EPT)], ew_v, sem0).wait()
    pltpu.make_async_copy(gi_hbm, gi_v, sem0).wait()
    pltpu.make_async_copy(gj_hbm, gj_v, sem0).wait()
    pltpu.make_async_copy(gc_hbm, gc_v, sem0).wait()
    pltpu.make_async_copy(zs_hbm, zero_v, sem0).wait()
    pltpu.make_async_copy(zd_hbm, degz_v, sem0).wait()
    pltpu.make_async_copy(on_hbm, ones_v, sem0).wait()

    with jax.named_scope("coef"):
        gcv = gc_v[pl.ds(0, 16)]
        gwe = _splat(gcv, 0)
        gb = _splat(gcv, 1)

        def _coef(i, c):
            for u in range(5):
                off = i * 80 + u * 16
                s16 = src_v[pl.ds(off, 16)]
                d16 = dst_v[pl.ds(off, 16)]
                gi = plsc.load_gather(gi_v, [d16])
                gj = plsc.load_gather(gj_v, [s16])
                w16 = ew_v[pl.ds(off, 16)]
                t = gi + gj + w16 * gwe + gb
                lamb = 1.0 / (1.0 + jnp.exp(-t))
                coeff_v[pl.ds(off, 16)] = w16 * lamb
            return c
        lax.fori_loop(0, EPT // 80, _coef, 0)

    row0 = sid * OWN
    nzc = jnp.where(sid == 15, (N - 15 * OWN) // ZR, OWN // ZR)

    for p in range(2):
        x_hbm = xlo_hbm if p == 0 else xhi_hbm

        with jax.named_scope(f"zero{p}"):
            def _zs(j, c):
                pltpu.sync_copy(zero_v, S_sh.at[pl.ds(row0 + j * ZR, ZR)])
                if p == 0:
                    pltpu.sync_copy(degz_v,
                                    deg_sh.at[pl.ds(row0 + j * ZR, ZR)])
                return c
            lax.fori_loop(0, nzc, _zs, 0)

            plsc.subcore_barrier()

        def _fill_src(slot, ch):
            eoff = ch * K
            for g in range(K // 16):
                srcc_v[slot, pl.ds(g * 16, 16)] = (
                    src_v[pl.ds(eoff + g * 16, 16)])

        def _fill_dst(slot, ch):
            eoff = ch * K
            for g in range(K // 16):
                dstc_v[slot, pl.ds(g * 16, 16)] = (
                    dst_v[pl.ds(eoff + g * 16, 16)])

        def _gather(slot, ch):
            _fill_src(slot, ch)
            pltpu.async_copy(x_hbm.at[srcc_v.at[slot]], rowsb_l[slot],
                             gsems[slot])

        def _gwait(slot):
            pltpu.make_async_copy(x_hbm.at[srcc_v.at[slot]],
                                  rowsb_l[slot], gsems[slot]).wait()

        def _scale(slot, ch):
            eoff = ch * K
            for g in range(K // 16):
                c16 = coeff_v[pl.ds(eoff + g * 16, 16)]
                for l in range(16):
                    e = g * 16 + l
                    sp = _splat(c16, l)
                    for c2 in range(H // 32):
                        b32 = rowsb_l[slot][e, pl.ds(c2 * 32, 32)]
                        v0, v1 = plsc.unpack(
                            b32, format=plsc.PackFormat.INTERLEAVED)
                        rows_l[slot][e, pl.ds(c2 * 32, 16)] = v0 * sp
                        rows_l[slot][e, pl.ds(c2 * 32 + 16, 16)] = v1 * sp

        def _sadd(slot, ch):
            _fill_dst(slot, ch)
            pltpu.async_copy(rows_l[slot], S_sh.at[dstc_v.at[slot]],
                             ssems[slot], add=True)
            if p == 0:
                pltpu.async_copy(ones_v, deg_sh.at[dstc_v.at[slot]],
                                 ssems[slot], add=True)

        def _sdrain(slot):
            pltpu.make_async_copy(rows_l[slot],
                                  S_sh.at[dstc_v.at[slot]],
                                  ssems[slot]).wait()
            if p == 0:
                pltpu.make_async_copy(ones_v, deg_sh.at[dstc_v.at[slot]],
                                      ssems[slot]).wait()

        sc_main = jax.named_scope(f"main{p}")
        sc_main.__enter__()
        for s in range(2):
            _gather(s, s)

        def _duo(i, c):
            c0 = 2 * i
            for s in range(2):
                @pl.when(i > 0)
                def _():
                    _sdrain(s)
                _gwait(s)
                _scale(s, c0 + s)
                _sadd(s, c0 + s)

                @pl.when(c0 + s + 2 < NCH)
                def _():
                    _gather(s, c0 + s + 2)
            return c
        lax.fori_loop(0, NCH // 2, _duo, 0)

        ls = (NCH - 1) % 2
        _sdrain(ls)
        _gwait(ls)
        _scale(ls, NCH - 1)
        _sadd(ls, NCH - 1)
        for s in range(2):
            _sdrain(s)
        sc_main.__exit__(None, None, None)

        with jax.named_scope(f"wout{p}"):
            plsc.subcore_barrier()

            def _wo(j, c):
                r = row0 + j * ZR
                pltpu.sync_copy(S_sh.at[pl.ds(r, ZR)],
                                s_out.at[cid, p, pl.ds(r, ZR)])
                if p == 0:
                    pltpu.sync_copy(deg_sh.at[pl.ds(r, ZR)],
                                    deg_out.at[cid, pl.ds(r, ZR)])
                return c
            lax.fori_loop(0, nzc, _wo, 0)

            if p == 0:
                plsc.subcore_barrier()


def _sc_aggregate(xlo, xhi, src, dst, ew, gi, gj, gc, zs, zd, on):
    mesh = plsc.VectorSubcoreMesh(core_axis_name="c", subcore_axis_name="s")
    f = functools.partial(
        pl.kernel,
        mesh=mesh,
        compiler_params=pltpu.CompilerParams(needs_layout_passes=False,
                                             use_tc_tiling_on_sc=False),
        out_type=[
            jax.ShapeDtypeStruct((2, 2, N, H), jnp.float32),
            jax.ShapeDtypeStruct((2, N, 8), jnp.float32),
        ],
        scratch_types=[
            pltpu.VMEM((EPT,), jnp.int32),
            pltpu.VMEM((EPT,), jnp.int32),
            pltpu.VMEM((EPT,), jnp.float32),
            pltpu.VMEM((EPT,), jnp.float32),
            pltpu.VMEM((N,), jnp.float32),
            pltpu.VMEM((N,), jnp.float32),
            pltpu.VMEM((16,), jnp.float32),
            pltpu.VMEM((K, H), jnp.float32),
            pltpu.VMEM((K, H), jnp.float32),
            pltpu.VMEM((K, H), jnp.bfloat16),
            pltpu.VMEM((K, H), jnp.bfloat16),
            pltpu.VMEM((2, K), jnp.int32),
            pltpu.VMEM((2, K), jnp.int32),
            pltpu.VMEM((K, 8), jnp.float32),
            pltpu.VMEM((ZR, H), jnp.float32),
            pltpu.VMEM((ZR, 8), jnp.float32),
            pltpu.VMEM_SHARED((N, H), jnp.float32),
            pltpu.VMEM_SHARED((N, 8), jnp.float32),
            pltpu.SemaphoreType.DMA,
            pltpu.SemaphoreType.DMA,
            pltpu.SemaphoreType.DMA,
            pltpu.SemaphoreType.DMA,
        ],
    )(_sc_body)
    return f(xlo, xhi, src, dst, ew, gi, gj, gc, zs, zd, on)


def _final_body(x_ref, s_ref, d_ref, amp_ref, w_ref, b_ref, o_ref):
    s_lo = s_ref[0, 0] + s_ref[1, 0]
    s_hi = s_ref[0, 1] + s_ref[1, 1]
    s = jnp.concatenate([s_lo, s_hi], axis=-1)
    dg = jnp.maximum(d_ref[0, :, 0:1] + d_ref[1, :, 0:1], 1.0)
    aggr = s * amp_ref[...] / dg
    o_ref[...] = (jnp.dot(x_ref[...], w_ref[0:D, :],
                          preferred_element_type=jnp.float32)
                  + jnp.dot(aggr, w_ref[D:2 * D, :],
                            preferred_element_type=jnp.float32)
                  + b_ref[...])


def _final(x2, s_parts, deg_parts, amp_weight, sage_w, sage_b2):
    blk = 400
    return pl.pallas_call(
        _final_body,
        grid=(N // blk,),
        in_specs=[
            pl.BlockSpec((blk, D), lambda i: (i, 0)),
            pl.BlockSpec((2, 2, blk, H), lambda i: (0, 0, i, 0)),
            pl.BlockSpec((2, blk, 8), lambda i: (0, i, 0)),
            pl.BlockSpec((1, D), lambda i: (0, 0)),
            pl.BlockSpec((2 * D, OUT), lambda i: (0, 0)),
            pl.BlockSpec((1, OUT), lambda i: (0, 0)),
        ],
        out_specs=pl.BlockSpec((blk, OUT), lambda i: (i, 0)),
        out_shape=jax.ShapeDtypeStruct((N, OUT), jnp.float32),
    )(x2, s_parts, deg_parts, amp_weight, sage_w, sage_b2)


def kernel(X, edge_index, edge_weight, amp_weight, gate_w, gate_b, sage_w,
           sage_b):
    x2 = X[0]
    gw2 = jnp.stack([gate_w[:D, 0], gate_w[D:2 * D, 0]], axis=1)
    gc = jnp.zeros((16,), jnp.float32)
    gc = gc.at[0].set(gate_w[2 * D, 0]).at[1].set(gate_b[0])

    g2 = _gate_proj(x2, gw2)
    gi = g2[:, 0]
    gj = g2[:, 1]

    def _prep(xh):
        a = xh.reshape(N, H // 32, 2, 16)
        return a.transpose(0, 1, 3, 2).reshape(N, H).astype(jnp.bfloat16)

    xlo = _prep(x2[:, :H])
    xhi = _prep(x2[:, H:])
    zs = jnp.zeros((ZR, H), jnp.float32)
    zd = jnp.zeros((ZR, 8), jnp.float32)
    on = jnp.zeros((K, 8), jnp.float32).at[:, 0].set(1.0)
    s_parts, deg_parts = _sc_aggregate(xlo, xhi, edge_index[0],
                                       edge_index[1], edge_weight,
                                       gi, gj, gc, zs, zd, on)

    out2 = _final(x2, s_parts, deg_parts, amp_weight,
                  sage_w, sage_b.reshape(1, OUT))
    return out2[None]


# speedup vs baseline: 1.1852x; 1.0535x over previous
import functools

import jax
import jax.numpy as jnp
from jax import lax
from jax.experimental import pallas as pl
from jax.experimental.pallas import tpu as pltpu
from jax.experimental.pallas import tpu_sc as plsc

N = 10000
E = 320000
D = 128
H = D // 2
OUT = 128

NW = 32
EPT = E // NW
K = 80
NCH = EPT // K
OWN = 640
ZR = 80


def _splat(vec16, lane):
    idx = jnp.full((16, 1), lane, dtype=jnp.int32)
    return lax.gather(
        vec16, idx,
        lax.GatherDimensionNumbers(offset_dims=(), collapsed_slice_dims=(0,),
                                   start_index_map=(0,)),
        (1,), mode=lax.GatherScatterMode.PROMISE_IN_BOUNDS)


def _gate_body(x_ref, w_ref, o_ref):
    o_ref[...] = jnp.dot(x_ref[...], w_ref[...],
                         preferred_element_type=jnp.float32)


def _gate_proj(x2, gw2):
    blk = 2000
    return pl.pallas_call(
        _gate_body,
        grid=(N // blk,),
        in_specs=[
            pl.BlockSpec((blk, D), lambda i: (i, 0)),
            pl.BlockSpec((D, 2), lambda i: (0, 0)),
        ],
        out_specs=pl.BlockSpec((blk, 2), lambda i: (i, 0)),
        out_shape=jax.ShapeDtypeStruct((N, 2), jnp.float32),
    )(x2, gw2)


def _sc_body(xb_hbm, ei_hbm, ew_hbm, gi_hbm, gj_hbm,
             gc_hbm, zs_hbm, zd_hbm, on_hbm, s_out, deg_out,
             src_v, dst_v, ew_v, coeff_v, gi_v, gj_v, gc_v,
             rw0, rw1, rb0, rb1, srcc_v, dstc_v, ones_v,
             zero_v, degz_v, S_sh, deg_sh,
             gs0, gs1, ss0, ss1):
    gsems = [gs0, gs1]
    ssems = [ss0, ss1]
    rows_l = [rw0, rw1]
    rowsb_l = [rb0, rb1]
    cid = lax.axis_index("c")
    sid = lax.axis_index("s")
    wid = sid * 2 + cid
    ebase = wid * EPT

    sem0 = gsems[0]
    pltpu.async_copy(ei_hbm.at[0, pl.ds(ebase, EPT)], src_v, sem0)
    pltpu.async_copy(ei_hbm.at[1, pl.ds(ebase, EPT)], dst_v, sem0)
    pltpu.async_copy(ew_hbm.at[pl.ds(ebase, EPT)], ew_v, sem0)
    pltpu.async_copy(gi_hbm, gi_v, sem0)
    pltpu.async_copy(gj_hbm, gj_v, sem0)
    pltpu.async_copy(gc_hbm, gc_v, sem0)
    pltpu.async_copy(zs_hbm, zero_v, sem0)
    pltpu.async_copy(zd_hbm, degz_v, sem0)
    pltpu.async_copy(on_hbm, ones_v, sem0)

    pltpu.make_async_copy(ei_hbm.at[0, pl.ds(ebase, EPT)], src_v,
                          sem0).wait()
    pltpu.make_async_copy(ei_hbm.at[1, pl.ds(ebase, EPT)], dst_v,
                          sem0).wait()
    pltpu.make_async_copy(ew_hbm.at[pl.ds(ebase, EPT)], ew_v, sem0).wait()
    pltpu.make_async_copy(gi_hbm, gi_v, sem0).wait()
    pltpu.make_async_copy(gj_hbm, gj_v, sem0).wait()
    pltpu.make_async_copy(gc_hbm, gc_v, sem0).wait()
    pltpu.make_async_copy(zs_hbm, zero_v, sem0).wait()
    pltpu.make_async_copy(zd_hbm, degz_v, sem0).wait()
    pltpu.make_async_copy(on_hbm, ones_v, sem0).wait()

    with jax.named_scope("coef"):
        gcv = gc_v[pl.ds(0, 16)]
        gwe = _splat(gcv, 0)
        gb = _splat(gcv, 1)

        def _coef(i, c):
            for u in range(5):
                off = i * 80 + u * 16
                s16 = src_v[pl.ds(off, 16)]
                d16 = dst_v[pl.ds(off, 16)]
                gi = plsc.load_gather(gi_v, [d16])
                gj = plsc.load_gather(gj_v, [s16])
                w16 = ew_v[pl.ds(off, 16)]
                t = gi + gj + w16 * gwe + gb
                lamb = 1.0 / (1.0 + jnp.exp(-t))
                coeff_v[pl.ds(off, 16)] = w16 * lamb
            return c
        lax.fori_loop(0, EPT // 80, _coef, 0)

    row0 = sid * OWN
    nzc = jnp.where(sid == 15, (N - 15 * OWN) // ZR, OWN // ZR)

    for p in range(2):
        x_hbm = xb_hbm.at[p]

        with jax.named_scope(f"zero{p}"):
            def _zs(j, c):
                pltpu.sync_copy(zero_v, S_sh.at[pl.ds(row0 + j * ZR, ZR)])
                if p == 0:
                    pltpu.sync_copy(degz_v,
                                    deg_sh.at[pl.ds(row0 + j * ZR, ZR)])
                return c
            lax.fori_loop(0, nzc, _zs, 0)

            plsc.subcore_barrier()

        def _fill_src(slot, ch):
            eoff = ch * K
            for g in range(K // 16):
                srcc_v[slot, pl.ds(g * 16, 16)] = (
                    src_v[pl.ds(eoff + g * 16, 16)])

        def _fill_dst(slot, ch):
            eoff = ch * K
            for g in range(K // 16):
                dstc_v[slot, pl.ds(g * 16, 16)] = (
                    dst_v[pl.ds(eoff + g * 16, 16)])

        def _gather(slot, ch):
            _fill_src(slot, ch)
            pltpu.async_copy(x_hbm.at[srcc_v.at[slot]], rowsb_l[slot],
                             gsems[slot])

        def _gwait(slot):
            pltpu.make_async_copy(x_hbm.at[srcc_v.at[slot]],
                                  rowsb_l[slot], gsems[slot]).wait()

        def _scale(slot, ch):
            eoff = ch * K
            for g in range(K // 16):
                c16 = coeff_v[pl.ds(eoff + g * 16, 16)]
                for l in range(16):
                    e = g * 16 + l
                    sp = _splat(c16, l)
                    for c2 in range(H // 32):
                        b32 = rowsb_l[slot][e, pl.ds(c2 * 32, 32)]
                        v0, v1 = plsc.unpack(
                            b32, format=plsc.PackFormat.INTERLEAVED)
                        rows_l[slot][e, pl.ds(c2 * 32, 16)] = v0 * sp
                        rows_l[slot][e, pl.ds(c2 * 32 + 16, 16)] = v1 * sp

        def _sadd(slot, ch):
            _fill_dst(slot, ch)
            pltpu.async_copy(rows_l[slot], S_sh.at[dstc_v.at[slot]],
                             ssems[slot], add=True)
            if p == 0:
                pltpu.async_copy(ones_v, deg_sh.at[dstc_v.at[slot]],
                                 ssems[slot], add=True)

        def _sdrain(slot):
            pltpu.make_async_copy(rows_l[slot],
                                  S_sh.at[dstc_v.at[slot]],
                                  ssems[slot]).wait()
            if p == 0:
                pltpu.make_async_copy(ones_v, deg_sh.at[dstc_v.at[slot]],
                                      ssems[slot]).wait()

        sc_main = jax.named_scope(f"main{p}")
        sc_main.__enter__()
        for s in range(2):
            _gather(s, s)

        def _duo(i, c):
            c0 = 2 * i
            for s in range(2):
                @pl.when(i > 0)
                def _():
                    _sdrain(s)
                _gwait(s)
                _scale(s, c0 + s)
                _sadd(s, c0 + s)

                @pl.when(c0 + s + 2 < NCH)
                def _():
                    _gather(s, c0 + s + 2)
            return c
        lax.fori_loop(0, NCH // 2, _duo, 0)

        ls = (NCH - 1) % 2
        _sdrain(ls)
        _gwait(ls)
        _scale(ls, NCH - 1)
        _sadd(ls, NCH - 1)
        for s in range(2):
            _sdrain(s)
        sc_main.__exit__(None, None, None)

        with jax.named_scope(f"wout{p}"):
            plsc.subcore_barrier()

            def _wo(j, c):
                r = row0 + j * ZR
                pltpu.sync_copy(S_sh.at[pl.ds(r, ZR)],
                                s_out.at[cid, p, pl.ds(r, ZR)])
                if p == 0:
                    pltpu.sync_copy(deg_sh.at[pl.ds(r, ZR)],
                                    deg_out.at[cid, pl.ds(r, ZR)])
                return c
            lax.fori_loop(0, nzc, _wo, 0)

            if p == 0:
                plsc.subcore_barrier()


def _sc_aggregate(xb, ei, ew, gi, gj, gc, zs, zd, on):
    mesh = plsc.VectorSubcoreMesh(core_axis_name="c", subcore_axis_name="s")
    f = functools.partial(
        pl.kernel,
        mesh=mesh,
        compiler_params=pltpu.CompilerParams(needs_layout_passes=False,
                                             use_tc_tiling_on_sc=False),
        out_type=[
            jax.ShapeDtypeStruct((2, 2, N, H), jnp.float32),
            jax.ShapeDtypeStruct((2, N, 8), jnp.float32),
        ],
        scratch_types=[
            pltpu.VMEM((EPT,), jnp.int32),
            pltpu.VMEM((EPT,), jnp.int32),
            pltpu.VMEM((EPT,), jnp.float32),
            pltpu.VMEM((EPT,), jnp.float32),
            pltpu.VMEM((N,), jnp.float32),
            pltpu.VMEM((N,), jnp.float32),
            pltpu.VMEM((16,), jnp.float32),
            pltpu.VMEM((K, H), jnp.float32),
            pltpu.VMEM((K, H), jnp.float32),
            pltpu.VMEM((K, H), jnp.bfloat16),
            pltpu.VMEM((K, H), jnp.bfloat16),
            pltpu.VMEM((2, K), jnp.int32),
            pltpu.VMEM((2, K), jnp.int32),
            pltpu.VMEM((K, 8), jnp.float32),
            pltpu.VMEM((ZR, H), jnp.float32),
            pltpu.VMEM((ZR, 8), jnp.float32),
            pltpu.VMEM_SHARED((N, H), jnp.float32),
            pltpu.VMEM_SHARED((N, 8), jnp.float32),
            pltpu.SemaphoreType.DMA,
            pltpu.SemaphoreType.DMA,
            pltpu.SemaphoreType.DMA,
            pltpu.SemaphoreType.DMA,
        ],
    )(_sc_body)
    return f(xb, ei, ew, gi, gj, gc, zs, zd, on)


def _final_body(x_ref, s_ref, d_ref, amp_ref, w_ref, b_ref, o_ref):
    s_lo = s_ref[0, 0] + s_ref[1, 0]
    s_hi = s_ref[0, 1] + s_ref[1, 1]
    s = jnp.concatenate([s_lo, s_hi], axis=-1)
    dg = jnp.maximum(d_ref[0, :, 0:1] + d_ref[1, :, 0:1], 1.0)
    aggr = s * amp_ref[...] / dg
    o_ref[...] = (jnp.dot(x_ref[...], w_ref[0:D, :],
                          preferred_element_type=jnp.float32)
                  + jnp.dot(aggr, w_ref[D:2 * D, :],
                            preferred_element_type=jnp.float32)
                  + b_ref[...])


def _final(x2, s_parts, deg_parts, amp_weight, sage_w, sage_b2):
    blk = 400
    return pl.pallas_call(
        _final_body,
        grid=(N // blk,),
        in_specs=[
            pl.BlockSpec((blk, D), lambda i: (i, 0)),
            pl.BlockSpec((2, 2, blk, H), lambda i: (0, 0, i, 0)),
            pl.BlockSpec((2, blk, 8), lambda i: (0, i, 0)),
            pl.BlockSpec((1, D), lambda i: (0, 0)),
            pl.BlockSpec((2 * D, OUT), lambda i: (0, 0)),
            pl.BlockSpec((1, OUT), lambda i: (0, 0)),
        ],
        out_specs=pl.BlockSpec((blk, OUT), lambda i: (i, 0)),
        out_shape=jax.ShapeDtypeStruct((N, OUT), jnp.float32),
    )(x2, s_parts, deg_parts, amp_weight, sage_w, sage_b2)


def kernel(X, edge_index, edge_weight, amp_weight, gate_w, gate_b, sage_w,
           sage_b):
    x2 = X[0]
    gw2 = jnp.stack([gate_w[:D, 0], gate_w[D:2 * D, 0]], axis=1)
    gc = jnp.zeros((16,), jnp.float32)
    gc = gc.at[0].set(gate_w[2 * D, 0]).at[1].set(gate_b[0])

    g2 = _gate_proj(x2, gw2)
    gi = g2[:, 0]
    gj = g2[:, 1]

    xb = (x2.reshape(N, 2, H // 32, 2, 16)
          .transpose(1, 0, 2, 4, 3)
          .reshape(2, N, H)
          .astype(jnp.bfloat16))
    zs = jnp.zeros((ZR, H), jnp.float32)
    zd = jnp.zeros((ZR, 8), jnp.float32)
    on = jnp.zeros((K, 8), jnp.float32).at[:, 0].set(1.0)
    s_parts, deg_parts = _sc_aggregate(xb, edge_index, edge_weight,
                                       gi, gj, gc, zs, zd, on)

    out2 = _final(x2, s_parts, deg_parts, amp_weight,
                  sage_w, sage_b.reshape(1, OUT))
    return out2[None]
